# Initial kernel scaffold; baseline (speedup 1.0000x reference)
import jax
import jax.numpy as jnp
from jax.experimental import pallas as pl


def kernel(x, table):
    raise NotImplementedError("write your pallas kernel here")


# speedup vs baseline: 4.0957x; 4.0957x over previous
import functools

import jax
import jax.numpy as jnp
from jax import lax
from jax.experimental import pallas as pl
from jax.experimental.pallas import tpu as pltpu
from jax.experimental.pallas import tpu_sc as plsc

NUM_ROWS = 100000
DIM = 64
BATCH = 4096 * 50
NC = 2
NS = 16
NW = NC * NS
BPW = BATCH // NW
CHUNK = 128
NCHUNK = BPW // CHUNK

_mesh = plsc.VectorSubcoreMesh(core_axis_name="c", subcore_axis_name="s")


@functools.partial(
    pl.kernel,
    mesh=_mesh,
    compiler_params=pltpu.CompilerParams(use_tc_tiling_on_sc=False),
    out_type=jax.ShapeDtypeStruct((BATCH, DIM), jnp.float32),
    scratch_types=[
        pltpu.VMEM((NCHUNK, CHUNK), jnp.int32),
        pltpu.VMEM((CHUNK, DIM), jnp.float32),
        pltpu.SemaphoreType.DMA,
    ],
)
def _embed_gather(idx_hbm, table_hbm, out_hbm, idx_v, rows_v, gsem):
    wid = lax.axis_index("s") * NC + lax.axis_index("c")
    base = wid * BPW
    pltpu.sync_copy(idx_hbm.at[wid], idx_v)

    def body(c, carry):
        pltpu.async_copy(table_hbm.at[idx_v.at[c]], rows_v, gsem).wait()
        pltpu.sync_copy(rows_v, out_hbm.at[pl.ds(base + c * CHUNK, CHUNK)])
        return carry

    lax.fori_loop(0, NCHUNK, body, 0)


def kernel(x, table):
    idx = x.reshape(NW, NCHUNK, CHUNK).astype(jnp.int32)
    out = _embed_gather(idx, table)
    return out.reshape(x.shape[0], x.shape[1], DIM)


# speedup vs baseline: 4.6776x; 1.1421x over previous
import functools

import jax
import jax.numpy as jnp
from jax import lax
from jax.experimental import pallas as pl
from jax.experimental.pallas import tpu as pltpu
from jax.experimental.pallas import tpu_sc as plsc

NUM_ROWS = 100000
DIM = 64
BATCH = 4096 * 50
NC = 2
NS = 16
NW = NC * NS
BPW = BATCH // NW
CHUNK = 128
NCHUNK = BPW // CHUNK
NB = 5

_mesh = plsc.VectorSubcoreMesh(core_axis_name="c", subcore_axis_name="s")


@functools.partial(
    pl.kernel,
    mesh=_mesh,
    compiler_params=pltpu.CompilerParams(use_tc_tiling_on_sc=False),
    out_type=jax.ShapeDtypeStruct((BATCH, DIM), jnp.float32),
    scratch_types=(
        [pltpu.VMEM((NCHUNK, CHUNK), jnp.int32)]
        + [pltpu.VMEM((NB, CHUNK, DIM), jnp.float32)]
        + [pltpu.SemaphoreType.DMA] * (2 * NB)
    ),
)
def _embed_gather(idx_hbm, table_hbm, out_hbm, idx_v, rows_v, *sems):
    gsem = sems[:NB]
    osem = sems[NB:]
    wid = lax.axis_index("s") * NC + lax.axis_index("c")
    base = wid * BPW
    pltpu.sync_copy(idx_hbm.at[wid], idx_v)

    def gather(c, p):
        return pltpu.make_async_copy(
            table_hbm.at[idx_v.at[c]], rows_v.at[p], gsem[p])

    def write(c, p):
        return pltpu.make_async_copy(
            rows_v.at[p], out_hbm.at[pl.ds(base + c * CHUNK, CHUNK)], osem[p])

    def step(c, j, first, fire):
        pp = (j - 1) % NB
        if not first:
            write(c - 1, pp).wait()
        if fire:
            gather(c + NB - 1, pp).start()
        gather(c, j).wait()
        write(c, j).start()

    for g in range(NB - 1):
        gather(g, g).start()

    for j in range(NB):
        step(j, j, first=(j == 0), fire=True)

    def body(i, carry):
        c0 = i * NB
        for j in range(NB):
            step(c0 + j, j, first=False, fire=True)
        return carry

    lax.fori_loop(1, NCHUNK // NB - 1, body, 0)

    c0 = NCHUNK - NB
    for j in range(NB):
        step(c0 + j, j, first=False, fire=(j == 0))

    write(NCHUNK - 1, (NCHUNK - 1) % NB).wait()


def kernel(x, table):
    idx = x.reshape(NW, NCHUNK, CHUNK).astype(jnp.int32)
    out = _embed_gather(idx, table)
    return out.reshape(x.shape[0], x.shape[1], DIM)
